# baseline (speedup 1.0000x reference)
import functools
import jax
import jax.numpy as jnp
from jax import lax
from jax.experimental import pallas as pl
from jax.experimental.pallas import tpu as pltpu
from jax.experimental.pallas import tpu_sc as plsc

D = 1024
B = 4
S = 4096
N = B * S
NW = 32
RPW = N // NW
C = 32
NCHUNK = RPW // C
LANES = 16
DCH = D // LANES
SCALE = 32.0


def _sc_body(x_hbm, tok_hbm, pos_hbm, out_hbm, idx_v, tok_v, pos_v, sem):
    cid = lax.axis_index("c")
    sid = lax.axis_index("s")
    wid = sid * 2 + cid
    base = wid * RPW
    s0 = base % S

    def chunk_body(g, carry):
        row0 = base + g * C
        p0 = s0 + g * C
        pltpu.sync_copy(x_hbm.at[pl.ds(row0, C)], idx_v)
        gat = pltpu.async_copy(tok_hbm.at[idx_v], tok_v, sem)
        pltpu.sync_copy(pos_hbm.at[pl.ds(p0, C)], pos_v)
        gat.wait()

        def row_body(r, rc):
            for d in range(DCH):
                sl = pl.ds(d * LANES, LANES)
                tok_v[r, sl] = tok_v[r, sl] * SCALE + pos_v[r, sl]
            return rc

        lax.fori_loop(0, C, row_body, 0)
        pltpu.sync_copy(tok_v, out_hbm.at[pl.ds(row0, C)])
        return carry

    lax.fori_loop(0, NCHUNK, chunk_body, 0)


@jax.jit
def _run(x_flat, token_table, pos_table):
    mesh = plsc.VectorSubcoreMesh(core_axis_name="c", subcore_axis_name="s")
    k = pl.kernel(
        _sc_body,
        out_type=jax.ShapeDtypeStruct((N, D), jnp.float32),
        mesh=mesh,
        scratch_types=[
            pltpu.VMEM((C,), jnp.int32),
            pltpu.VMEM((C, D), jnp.float32),
            pltpu.VMEM((C, D), jnp.float32),
            pltpu.SemaphoreType.DMA,
        ],
    )
    return k(x_flat, token_table, pos_table)


def kernel(x, token_table, pos_table):
    out = _run(x.reshape(-1), token_table, pos_table)
    return out.reshape(B, S, D)


# speedup vs baseline: 1.6880x; 1.6880x over previous
import functools
import jax
import jax.numpy as jnp
from jax import lax
from jax.experimental import pallas as pl
from jax.experimental.pallas import tpu as pltpu
from jax.experimental.pallas import tpu_sc as plsc

D = 1024
B = 4
S = 4096
N = B * S
NW = 32
RPW = N // NW
C = 16
G = RPW // C
LANES = 16
DCH = D // LANES
SCALE = 32.0


def _sc_body(x_hbm, tok_hbm, pos_hbm, out_hbm,
             idxall, tok0, tok1, pos0, pos1, ob0, ob1,
             gs0, gs1, ps0, ps1, os0, os1):
    cid = lax.axis_index("c")
    sid = lax.axis_index("s")
    wid = sid * 2 + cid
    base = wid * RPW
    s0 = base % S

    pltpu.sync_copy(x_hbm.at[pl.ds(base, RPW)], idxall)

    toks = (tok0, tok1)
    poss = (pos0, pos1)
    obs = (ob0, ob1)
    gss = (gs0, gs1)
    pss = (ps0, ps1)
    oss = (os0, os1)

    def issue(g, bb):
        pltpu.async_copy(tok_hbm.at[idxall.at[pl.ds(g * C, C)]], toks[bb], gss[bb])
        pltpu.async_copy(pos_hbm.at[pl.ds(s0 + g * C, C)], poss[bb], pss[bb])

    def wait_in(g, bb):
        pltpu.make_async_copy(
            tok_hbm.at[idxall.at[pl.ds(g * C, C)]], toks[bb], gss[bb]).wait()
        pltpu.make_async_copy(
            pos_hbm.at[pl.ds(s0 + g * C, C)], poss[bb], pss[bb]).wait()

    def wait_out(bb):
        pltpu.make_async_copy(obs[bb], out_hbm.at[pl.ds(base, C)], oss[bb]).wait()

    issue(0, 0)

    def pair_body(i, carry):
        for bb in (0, 1):
            g = i * 2 + bb
            nb = 1 - bb
            if bb == 0:
                issue(g + 1, nb)
            else:
                @pl.when(i < (G // 2 - 1))
                def _():
                    issue(g + 1, nb)
            wait_in(g, bb)
            @pl.when(i >= 1)
            def _():
                wait_out(bb)
            tokb, posb, outb = toks[bb], poss[bb], obs[bb]

            def row(r, rc):
                for d in range(DCH):
                    sl = pl.ds(d * LANES, LANES)
                    outb[r, sl] = tokb[r, sl] * SCALE + posb[r, sl]
                return rc

            lax.fori_loop(0, C, row, 0)
            pltpu.async_copy(outb, out_hbm.at[pl.ds(base + g * C, C)], oss[bb])
        return carry

    lax.fori_loop(0, G // 2, pair_body, 0)
    wait_out(0)
    wait_out(1)


@jax.jit
def _run(x_flat, token_table, pos_table):
    mesh = plsc.VectorSubcoreMesh(core_axis_name="c", subcore_axis_name="s")
    k = pl.kernel(
        _sc_body,
        out_type=jax.ShapeDtypeStruct((N, D), jnp.float32),
        mesh=mesh,
        scratch_types=[
            pltpu.VMEM((RPW,), jnp.int32),
            pltpu.VMEM((C, D), jnp.float32),
            pltpu.VMEM((C, D), jnp.float32),
            pltpu.VMEM((C, D), jnp.float32),
            pltpu.VMEM((C, D), jnp.float32),
            pltpu.VMEM((C, D), jnp.float32),
            pltpu.VMEM((C, D), jnp.float32),
            pltpu.SemaphoreType.DMA,
            pltpu.SemaphoreType.DMA,
            pltpu.SemaphoreType.DMA,
            pltpu.SemaphoreType.DMA,
            pltpu.SemaphoreType.DMA,
            pltpu.SemaphoreType.DMA,
        ],
    )
    return k(x_flat, token_table, pos_table)


def kernel(x, token_table, pos_table):
    out = _run(x.reshape(-1), token_table, pos_table)
    return out.reshape(B, S, D)


# speedup vs baseline: 1.7429x; 1.0326x over previous
import functools
import jax
import jax.numpy as jnp
from jax import lax
from jax.experimental import pallas as pl
from jax.experimental.pallas import tpu as pltpu
from jax.experimental.pallas import tpu_sc as plsc

D = 1024
B = 4
S = 4096
N = B * S
NW = 32
RPW = N // NW
C = 8
G = RPW // C
NBUF = 4
LANES = 16
DCH = D // LANES
SCALE = 32.0


def _sc_body(x_hbm, tok_hbm, pos_hbm, out_hbm,
             idxall, tok0, tok1, tok2, tok3, pos0, pos1, pos2, pos3,
             gs0, gs1, gs2, gs3, ps0, ps1, ps2, ps3, os0, os1, os2, os3):
    cid = lax.axis_index("c")
    sid = lax.axis_index("s")
    wid = sid * 2 + cid
    base = wid * RPW
    s0 = base % S

    pltpu.sync_copy(x_hbm.at[pl.ds(base, RPW)], idxall)

    toks = (tok0, tok1, tok2, tok3)
    poss = (pos0, pos1, pos2, pos3)
    gss = (gs0, gs1, gs2, gs3)
    pss = (ps0, ps1, ps2, ps3)
    oss = (os0, os1, os2, os3)

    def issue(g, bb):
        pltpu.async_copy(tok_hbm.at[idxall.at[pl.ds(g * C, C)]], toks[bb], gss[bb])
        pltpu.async_copy(pos_hbm.at[pl.ds(s0 + g * C, C)], poss[bb], pss[bb])

    def wait_in(g, bb):
        pltpu.make_async_copy(
            tok_hbm.at[idxall.at[pl.ds(g * C, C)]], toks[bb], gss[bb]).wait()
        pltpu.make_async_copy(
            pos_hbm.at[pl.ds(s0 + g * C, C)], poss[bb], pss[bb]).wait()

    def wait_out(bb):
        pltpu.make_async_copy(poss[bb], out_hbm.at[pl.ds(base, C)], oss[bb]).wait()

    issue(0, 0)
    issue(1, 1)

    def quad_body(i, carry):
        for bb in range(NBUF):
            g = i * NBUF + bb
            b2 = (bb + 2) % NBUF
            if bb < 2:
                @pl.when(i >= 1)
                def _():
                    wait_out(b2)
                issue(g + 2, b2)
            else:
                wait_out(b2)

                @pl.when(i < (G // NBUF - 1))
                def _():
                    issue(g + 2, b2)
            wait_in(g, bb)
            tokb, posb = toks[bb], poss[bb]

            def row(r, rc):
                for d in range(DCH):
                    sl = pl.ds(d * LANES, LANES)
                    plsc.addupdate(posb.at[r, sl], tokb[r, sl] * SCALE)
                return rc

            lax.fori_loop(0, C, row, 0)
            pltpu.async_copy(posb, out_hbm.at[pl.ds(base + g * C, C)], oss[bb])
        return carry

    lax.fori_loop(0, G // NBUF, quad_body, 0)
    wait_out(2)
    wait_out(3)


@jax.jit
def _run(x_flat, token_table, pos_table):
    mesh = plsc.VectorSubcoreMesh(core_axis_name="c", subcore_axis_name="s")
    k = pl.kernel(
        _sc_body,
        out_type=jax.ShapeDtypeStruct((N, D), jnp.float32),
        mesh=mesh,
        scratch_types=(
            [pltpu.VMEM((RPW,), jnp.int32)]
            + [pltpu.VMEM((C, D), jnp.float32) for _ in range(2 * NBUF)]
            + [pltpu.SemaphoreType.DMA for _ in range(3 * NBUF)]
        ),
    )
    return k(x_flat, token_table, pos_table)


def kernel(x, token_table, pos_table):
    out = _run(x.reshape(-1), token_table, pos_table)
    return out.reshape(B, S, D)
